# baseline (speedup 1.0000x reference)
import math

import jax
import jax.numpy as jnp
from jax.experimental import pallas as pl
from jax.experimental.pallas import tpu as pltpu

_D_MODEL = 128
_N_HEADS = 8


def _make_body(d_model, n_heads, n_objs):
    rows_wo = ((n_heads + 7) // 8) * 8
    row_bbig = rows_wo + d_model
    row_bo = row_bbig + 1
    am_scale = math.sqrt(n_objs)

    def _body(x_ref, wbig_ref, consts_ref, out_ref, amc_ref):
        bb = x_ref.shape[0]
        d_embed = x_ref.shape[2]
        n = bb * n_objs

        x2 = x_ref[...].reshape(n, d_embed).astype(jnp.bfloat16)
        w = wbig_ref[...].astype(jnp.bfloat16)

        proj = jnp.dot(x2, w, preferred_element_type=jnp.float32)

        e = jnp.exp(proj[:, d_model:d_model + n_heads])
        e3 = e.reshape(bb, n_objs, n_heads)
        inv = pl.reciprocal(jnp.sum(e3, axis=1) + 1.0, approx=True)
        p3 = e3 * inv[:, None, :]
        pf = p3.reshape(n, n_heads)

        expand = consts_ref[0:n_heads, 0:d_model]
        e_exp = jnp.dot(pf, expand, preferred_element_type=jnp.float32)
        y = e_exp * proj[:, 0:d_model]
        bv = consts_ref[row_bbig:row_bbig + 1, 0:d_model]
        ctx = jnp.sum(y.reshape(bb, n_objs, d_model), axis=1) + bv

        wo = consts_ref[rows_wo:rows_wo + d_model, 0:d_model]
        bo = consts_ref[row_bo:row_bo + 1, 0:d_model]
        out_ref[...] = jnp.dot(ctx, wo, preferred_element_type=jnp.float32) + bo

        amc_ref[...] = pf[:, 0:1] * am_scale

    return _body


def kernel(obj_latents, wbig, consts):
    bs, n_objs, d_embed = obj_latents.shape
    d_model, n_heads = _D_MODEL, _N_HEADS
    P = wbig.shape[1]
    Rc = consts.shape[0]
    f32 = jnp.float32

    n_blocks = 1
    for nb in (16, 8, 4, 2):
        if bs % nb == 0:
            n_blocks = nb
            break
    bb = bs // n_blocks

    body = _make_body(d_model, n_heads, n_objs)
    rep = lambda b: (0, 0)
    out, am_col = pl.pallas_call(
        body,
        grid=(n_blocks,),
        in_specs=[
            pl.BlockSpec((bb, n_objs, d_embed), lambda b: (b, 0, 0)),
            pl.BlockSpec((d_embed, P), rep),
            pl.BlockSpec((Rc, P), rep),
        ],
        out_specs=[
            pl.BlockSpec((bb, d_model), lambda b: (b, 0)),
            pl.BlockSpec((bb * n_objs, 1), lambda b: (b, 0)),
        ],
        out_shape=[
            jax.ShapeDtypeStruct((bs, d_model), f32),
            jax.ShapeDtypeStruct((bs * n_objs, 1), f32),
        ],
        compiler_params=pltpu.CompilerParams(dimension_semantics=("parallel",)),
    )(obj_latents.astype(f32), wbig, consts)
    return out, am_col.reshape(bs, n_objs)[0:1, :]


# speedup vs baseline: 1.5631x; 1.5631x over previous
import math

import jax
import jax.numpy as jnp
from jax.experimental import pallas as pl
from jax.experimental.pallas import tpu as pltpu

_D_MODEL = 128
_N_HEADS = 8


def _make_body(d_model, n_heads, n_objs):
    rows_wo = ((n_heads + 7) // 8) * 8
    row_bbig = rows_wo + d_model
    row_bo = row_bbig + 1
    am_scale = math.sqrt(n_objs)

    def _body(x_ref, wbig_ref, consts_ref, out_ref, amc_ref):
        bb = x_ref.shape[0]
        d_embed = x_ref.shape[2]
        n = bb * n_objs

        x2 = x_ref[...].reshape(n, d_embed).astype(jnp.bfloat16)
        w = wbig_ref[...].astype(jnp.bfloat16)

        proj = jnp.dot(x2, w, preferred_element_type=jnp.float32)

        e = jnp.exp(proj[:, d_model:d_model + n_heads])
        e3 = e.reshape(bb, n_objs, n_heads)
        inv = pl.reciprocal(jnp.sum(e3, axis=1) + 1.0, approx=True)
        p3 = e3 * inv[:, None, :]
        pf = p3.reshape(n, n_heads)

        expand = consts_ref[0:n_heads, 0:d_model]
        e_exp = jnp.dot(pf, expand, preferred_element_type=jnp.float32)
        y = e_exp * proj[:, 0:d_model]
        bv = consts_ref[row_bbig:row_bbig + 1, 0:d_model]
        ctx = jnp.sum(y.reshape(bb, n_objs, d_model), axis=1) + bv

        wo = consts_ref[rows_wo:rows_wo + d_model, 0:d_model]
        bo = consts_ref[row_bo:row_bo + 1, 0:d_model]
        out_ref[...] = jnp.dot(ctx, wo, preferred_element_type=jnp.float32) + bo

        amc_ref[...] = pf[:, 0:1] * am_scale

    return _body


def kernel(obj_latents, wbig, consts):
    bs, n_objs, d_embed = obj_latents.shape
    d_model, n_heads = _D_MODEL, _N_HEADS
    P = wbig.shape[1]
    Rc = consts.shape[0]
    f32 = jnp.float32

    n_blocks = 1
    for nb in (4, 2):
        if bs % nb == 0:
            n_blocks = nb
            break
    bb = bs // n_blocks

    body = _make_body(d_model, n_heads, n_objs)
    rep = lambda b: (0, 0)
    out, am_col = pl.pallas_call(
        body,
        grid=(n_blocks,),
        in_specs=[
            pl.BlockSpec((bb, n_objs, d_embed), lambda b: (b, 0, 0)),
            pl.BlockSpec((d_embed, P), rep),
            pl.BlockSpec((Rc, P), rep),
        ],
        out_specs=[
            pl.BlockSpec((bb, d_model), lambda b: (b, 0)),
            pl.BlockSpec((bb * n_objs, 1), lambda b: (b, 0)),
        ],
        out_shape=[
            jax.ShapeDtypeStruct((bs, d_model), f32),
            jax.ShapeDtypeStruct((bs * n_objs, 1), f32),
        ],
        compiler_params=pltpu.CompilerParams(dimension_semantics=("parallel",)),
    )(obj_latents.astype(f32), wbig, consts)
    return out, am_col.reshape(bs, n_objs)[0:1, :]


# speedup vs baseline: 1.6294x; 1.0425x over previous
import math

import jax
import jax.numpy as jnp
from jax.experimental import pallas as pl
from jax.experimental.pallas import tpu as pltpu

_D_MODEL = 128
_N_HEADS = 8


def _make_body(d_model, n_heads, n_objs):
    rows_wo = ((n_heads + 7) // 8) * 8
    row_bbig = rows_wo + d_model
    row_bo = row_bbig + 1
    am_scale = math.sqrt(n_objs)

    def _body(x_ref, wbig_ref, consts_ref, out_ref, amc_ref):
        bb = x_ref.shape[0]
        d_embed = x_ref.shape[2]
        n = bb * n_objs

        x2 = x_ref[...].reshape(n, d_embed).astype(jnp.bfloat16)
        w = wbig_ref[...].astype(jnp.bfloat16)

        proj = jnp.dot(x2, w, preferred_element_type=jnp.float32)

        e = jnp.exp(proj[:, d_model:d_model + n_heads])
        e3 = e.reshape(bb, n_objs, n_heads)
        inv = pl.reciprocal(jnp.sum(e3, axis=1) + 1.0, approx=True)
        p3 = e3 * inv[:, None, :]
        pf = p3.reshape(n, n_heads)

        expand = consts_ref[0:n_heads, 0:d_model]
        e_exp = jnp.dot(pf, expand, preferred_element_type=jnp.float32)
        y = e_exp * proj[:, 0:d_model]
        bv = consts_ref[row_bbig:row_bbig + 1, 0:d_model]
        ctx = jnp.sum(y.reshape(bb, n_objs, d_model), axis=1) + bv

        wo = consts_ref[rows_wo:rows_wo + d_model, 0:d_model]
        bo = consts_ref[row_bo:row_bo + 1, 0:d_model]
        out_ref[...] = jnp.dot(ctx, wo, preferred_element_type=jnp.float32) + bo

        amc_ref[...] = pf[:, 0:1] * am_scale

    return _body


def kernel(obj_latents, wbig, consts):
    bs, n_objs, d_embed = obj_latents.shape
    d_model, n_heads = _D_MODEL, _N_HEADS
    P = wbig.shape[1]
    Rc = consts.shape[0]
    f32 = jnp.float32

    n_blocks = 1
    for nb in (2,):
        if bs % nb == 0:
            n_blocks = nb
            break
    bb = bs // n_blocks

    body = _make_body(d_model, n_heads, n_objs)
    rep = lambda b: (0, 0)
    out, am_col = pl.pallas_call(
        body,
        grid=(n_blocks,),
        in_specs=[
            pl.BlockSpec((bb, n_objs, d_embed), lambda b: (b, 0, 0)),
            pl.BlockSpec((d_embed, P), rep),
            pl.BlockSpec((Rc, P), rep),
        ],
        out_specs=[
            pl.BlockSpec((bb, d_model), lambda b: (b, 0)),
            pl.BlockSpec((bb * n_objs, 1), lambda b: (b, 0)),
        ],
        out_shape=[
            jax.ShapeDtypeStruct((bs, d_model), f32),
            jax.ShapeDtypeStruct((bs * n_objs, 1), f32),
        ],
        compiler_params=pltpu.CompilerParams(dimension_semantics=("parallel",)),
    )(obj_latents.astype(f32), wbig, consts)
    return out, am_col.reshape(bs, n_objs)[0:1, :]


# speedup vs baseline: 1.9426x; 1.1922x over previous
import math

import jax
import jax.numpy as jnp
from jax.experimental import pallas as pl
from jax.experimental.pallas import tpu as pltpu

_D_MODEL = 128
_N_HEADS = 8


def _make_body(d_model, n_heads, n_objs):
    rows_wo = ((n_heads + 7) // 8) * 8
    row_bbig = rows_wo + d_model
    row_bo = row_bbig + 1
    am_scale = math.sqrt(n_objs)

    def _body(x_ref, x0_ref, wbig_ref, consts_ref, out_ref, am_ref):
        bb = x_ref.shape[0]
        d_embed = x_ref.shape[2]
        n = bb * n_objs

        x2 = x_ref[...].reshape(n, d_embed).astype(jnp.bfloat16)
        w = wbig_ref[...].astype(jnp.bfloat16)

        proj = jnp.dot(x2, w, preferred_element_type=jnp.float32)

        e = jnp.exp(proj[:, d_model:d_model + n_heads])
        e3 = e.reshape(bb, n_objs, n_heads)
        inv = pl.reciprocal(jnp.sum(e3, axis=1) + 1.0, approx=True)
        p3 = e3 * inv[:, None, :]
        pf = p3.reshape(n, n_heads)

        expand = consts_ref[0:n_heads, 0:d_model]
        e_exp = jnp.dot(pf, expand, preferred_element_type=jnp.float32)
        y = e_exp * proj[:, 0:d_model]
        bv = consts_ref[row_bbig:row_bbig + 1, 0:d_model]
        ctx = jnp.sum(y.reshape(bb, n_objs, d_model), axis=1) + bv

        wo = consts_ref[rows_wo:rows_wo + d_model, 0:d_model]
        bo = consts_ref[row_bo:row_bo + 1, 0:d_model]
        out_ref[...] = jnp.dot(ctx, wo, preferred_element_type=jnp.float32) + bo

        x0 = x0_ref[...].reshape(n_objs, d_embed).astype(jnp.bfloat16)
        s0 = jnp.dot(x0, w[:, d_model:d_model + n_heads],
                     preferred_element_type=jnp.float32)
        e0 = jnp.exp(s0[:, 0:1]).reshape(1, n_objs)
        d0 = jnp.sum(e0, axis=1, keepdims=True) + 1.0
        am_ref[...] = e0 * (am_scale * pl.reciprocal(d0, approx=True))

    return _body


def kernel(obj_latents, wbig, consts):
    bs, n_objs, d_embed = obj_latents.shape
    d_model, n_heads = _D_MODEL, _N_HEADS
    P = wbig.shape[1]
    Rc = consts.shape[0]
    f32 = jnp.float32

    n_blocks = 2 if bs % 2 == 0 else 1
    bb = bs // n_blocks

    body = _make_body(d_model, n_heads, n_objs)
    rep = lambda b: (0, 0)
    out, am = pl.pallas_call(
        body,
        grid=(n_blocks,),
        in_specs=[
            pl.BlockSpec((bb, n_objs, d_embed), lambda b: (b, 0, 0)),
            pl.BlockSpec((1, n_objs, d_embed), lambda b: (0, 0, 0)),
            pl.BlockSpec((d_embed, P), rep),
            pl.BlockSpec((Rc, P), rep),
        ],
        out_specs=[
            pl.BlockSpec((bb, d_model), lambda b: (b, 0)),
            pl.BlockSpec((1, n_objs), rep),
        ],
        out_shape=[
            jax.ShapeDtypeStruct((bs, d_model), f32),
            jax.ShapeDtypeStruct((1, n_objs), f32),
        ],
        compiler_params=pltpu.CompilerParams(dimension_semantics=("parallel",)),
    )(obj_latents.astype(f32), obj_latents.astype(f32), wbig, consts)
    return out, am
